# Initial kernel scaffold; baseline (speedup 1.0000x reference)
import jax
import jax.numpy as jnp
from jax.experimental import pallas as pl


def kernel(feat, x, edge_index, tails, W1, b1, W2, b2, Wf, bf, Wc1, bc1, Wc2, bc2):
    raise NotImplementedError("write your pallas kernel here")


# speedup vs baseline: 26.9319x; 26.9319x over previous
import functools

import jax
import jax.numpy as jnp
from jax import lax
from jax.experimental import pallas as pl
from jax.experimental.pallas import tpu as pltpu
from jax.experimental.pallas import tpu_sc as plsc

N = 10000
E = 320000
D = 128
T = 1000
NUM_PRED = 5
HID = 64
M = N + T * NUM_PRED
MPAD = 15104
NW = 32
EPB = 128
NBLK = 80
EPT = EPB * NBLK
EPAD = NW * EPT
EB16 = EPT // 16
RPT = MPAD // 16


@functools.cache
def _make_sc_count():
    mesh = plsc.VectorSubcoreMesh(core_axis_name="c", subcore_axis_name="s")
    return functools.partial(
        pl.kernel,
        mesh=mesh,
        compiler_params=pltpu.CompilerParams(needs_layout_passes=False, use_tc_tiling_on_sc=False),
        out_type=jax.ShapeDtypeStruct((NW, MPAD), jnp.float32),
        scratch_types=[
            pltpu.VMEM((EB16, 16), jnp.int32),
            pltpu.VMEM((MPAD,), jnp.float32),
        ],
    )(_sc_count_body)


def _sc_count_body(dst_hbm, out_hbm, dst_v, acc_v):
    c = lax.axis_index("c")
    s = lax.axis_index("s")
    wid = s * 2 + c
    pltpu.sync_copy(dst_hbm.at[wid], dst_v)

    def zero(i, carry):
        acc_v[pl.ds(i * 16, 16)] = jnp.zeros((16,), jnp.float32)
        return carry

    lax.fori_loop(0, MPAD // 16, zero, 0)
    ones = jnp.full((16,), 1.0, jnp.float32)

    def body(i, carry):
        plsc.addupdate_scatter(acc_v, [dst_v[i]], ones)
        return carry

    lax.fori_loop(0, EB16, body, 0)
    pltpu.sync_copy(acc_v, out_hbm.at[wid])


@functools.cache
def _make_sc_scalar():
    mesh = plsc.VectorSubcoreMesh(core_axis_name="c", subcore_axis_name="s")
    return functools.partial(
        pl.kernel,
        mesh=mesh,
        compiler_params=pltpu.CompilerParams(needs_layout_passes=False, use_tc_tiling_on_sc=False),
        out_type=jax.ShapeDtypeStruct((NW, MPAD), jnp.float32),
        scratch_types=[
            pltpu.VMEM((EB16, 16), jnp.int32),
            pltpu.VMEM((EB16, 16), jnp.int32),
            pltpu.VMEM((MPAD,), jnp.float32),
            pltpu.VMEM((MPAD,), jnp.float32),
        ],
    )(_sc_scalar_body)


def _sc_scalar_body(g2_hbm, src_hbm, dst_hbm, out_hbm, src_v, dst_v, g2_v, acc_v):
    c = lax.axis_index("c")
    s = lax.axis_index("s")
    wid = s * 2 + c
    pltpu.sync_copy(src_hbm.at[wid], src_v)
    pltpu.sync_copy(dst_hbm.at[wid], dst_v)
    pltpu.sync_copy(g2_hbm, g2_v)

    def zero(i, carry):
        acc_v[pl.ds(i * 16, 16)] = jnp.zeros((16,), jnp.float32)
        return carry

    lax.fori_loop(0, MPAD // 16, zero, 0)

    def body(i, carry):
        vals = plsc.load_gather(g2_v, [src_v[i]])
        plsc.addupdate_scatter(acc_v, [dst_v[i]], vals)
        return carry

    lax.fori_loop(0, EB16, body, 0)
    pltpu.sync_copy(acc_v, out_hbm.at[wid])


@functools.cache
def _make_sc_rows():
    mesh = plsc.VectorSubcoreMesh(core_axis_name="c", subcore_axis_name="s")
    return functools.partial(
        pl.kernel,
        mesh=mesh,
        compiler_params=pltpu.CompilerParams(needs_layout_passes=False, use_tc_tiling_on_sc=False),
        out_type=jax.ShapeDtypeStruct((2, MPAD, HID), jnp.float32),
        scratch_types=[
            pltpu.VMEM((NBLK, EPB), jnp.int32),
            pltpu.VMEM((NBLK, EPB), jnp.int32),
            pltpu.VMEM((EPB, HID), jnp.float32),
            pltpu.SemaphoreType.DMA,
            pltpu.VMEM_SHARED((MPAD, HID), jnp.float32),
        ],
    )(_sc_rows_body)


def _sc_rows_body(g_hbm, zeros_hbm, src_hbm, dst_hbm, out_hbm,
                  src_v, dst_v, rows_v, sem, acc_sh):
    c = lax.axis_index("c")
    s = lax.axis_index("s")
    wid = s * 2 + c
    pltpu.sync_copy(src_hbm.at[wid], src_v)
    pltpu.sync_copy(dst_hbm.at[wid], dst_v)
    base = s * RPT
    pltpu.sync_copy(zeros_hbm.at[pl.ds(base, RPT)], acc_sh.at[pl.ds(base, RPT)])
    plsc.subcore_barrier()

    def body(j, carry):
        pltpu.async_copy(g_hbm.at[src_v.at[j]], rows_v, sem).wait()
        pltpu.sync_copy(rows_v, acc_sh.at[dst_v.at[j]], add=True)
        return carry

    lax.fori_loop(0, NBLK, body, 0)
    plsc.subcore_barrier()
    pltpu.sync_copy(acc_sh.at[pl.ds(base, RPT)], out_hbm.at[c, pl.ds(base, RPT)])


def _mlp_body(feat_ref, w1_ref, b1_ref, w2_ref, b2_ref, wf_ref, bf_ref, out_ref):
    h = jnp.dot(feat_ref[...], w1_ref[...], preferred_element_type=jnp.float32)
    h = jnp.maximum(h + b1_ref[...], 0.0)
    h = jnp.dot(h, w2_ref[...], preferred_element_type=jnp.float32)
    h = jnp.maximum(h + b2_ref[...], 0.0)
    h = jnp.dot(h, wf_ref[...], preferred_element_type=jnp.float32)
    out_ref[...] = jnp.tanh(h + bf_ref[...])


def _deg_body(parts_ref, out_ref):
    deg = jnp.sum(parts_ref[...], axis=0, keepdims=True) + 1.0
    out_ref[...] = lax.rsqrt(deg)


def _prep_body(nf_ref, wc1_ref, dinv_ref, g_ref):
    h = jnp.dot(nf_ref[...], wc1_ref[...], preferred_element_type=jnp.float32)
    g_ref[...] = h * dinv_ref[...]


def _mid_body(a0_ref, a1_ref, g_ref, dinv_ref, bc1_ref, wc2_ref, g2_ref):
    t = a0_ref[...] + a1_ref[...] + g_ref[...]
    out1 = jnp.maximum(t * dinv_ref[...] + bc1_ref[...], 0.0)
    s = jnp.sum(out1 * wc2_ref[...], axis=1, keepdims=True)
    g2_ref[...] = s * dinv_ref[...]


def _final_body(parts_ref, g2_ref, dinv_ref, bc2_ref, out_ref):
    acc2 = jnp.sum(parts_ref[...], axis=0, keepdims=True)
    out2 = dinv_ref[...] * (acc2 + g2_ref[...]) + bc2_ref[...]
    out_ref[...] = 1.0 / (1.0 + jnp.exp(-out2))


def kernel(feat, x, edge_index, tails, W1, b1, W2, b2, Wf, bf, Wc1, bc1, Wc2, bc2):
    npad = EPAD - E - T * NUM_PRED
    src = jnp.concatenate([
        edge_index[0],
        jnp.repeat(tails, NUM_PRED),
        jnp.zeros((npad,), jnp.int32),
    ])
    dst = jnp.concatenate([
        edge_index[1],
        jnp.arange(N, M, dtype=jnp.int32),
        jnp.full((npad,), MPAD - 1, jnp.int32),
    ])
    src_r = src.reshape(NW, NBLK, EPB)
    dst_r = dst.reshape(NW, NBLK, EPB)
    src_s = src.reshape(NW, EB16, 16)
    dst_s = dst.reshape(NW, EB16, 16)

    parts = _make_sc_count()(dst_s)

    dinv_row = pl.pallas_call(
        _deg_body,
        out_shape=jax.ShapeDtypeStruct((1, MPAD), jnp.float32),
    )(parts)
    dinv_col = dinv_row.reshape(MPAD, 1)

    gen_feat = pl.pallas_call(
        _mlp_body,
        out_shape=jax.ShapeDtypeStruct((T, NUM_PRED * D), jnp.float32),
    )(feat, W1, b1.reshape(1, -1), W2, b2.reshape(1, -1), Wf, bf.reshape(1, -1))

    new_feat = jnp.concatenate([
        x,
        gen_feat.reshape(T * NUM_PRED, D),
        jnp.zeros((MPAD - M, D), jnp.float32),
    ], axis=0)

    g = pl.pallas_call(
        _prep_body,
        out_shape=jax.ShapeDtypeStruct((MPAD, HID), jnp.float32),
    )(new_feat, Wc1, dinv_col)

    zeros = jnp.zeros((MPAD, HID), jnp.float32)
    acc = _make_sc_rows()(g, zeros, src_r, dst_r)

    g2_col = pl.pallas_call(
        _mid_body,
        out_shape=jax.ShapeDtypeStruct((MPAD, 1), jnp.float32),
    )(acc[0], acc[1], g, dinv_col, bc1.reshape(1, -1), Wc2.reshape(1, -1))

    parts2 = _make_sc_scalar()(g2_col.reshape(MPAD), src_s, dst_s)

    pred_row = pl.pallas_call(
        _final_body,
        out_shape=jax.ShapeDtypeStruct((1, MPAD), jnp.float32),
    )(parts2, g2_col.reshape(1, MPAD), dinv_row, bc2.reshape(1, 1))

    class_pred = pred_row.reshape(MPAD, 1)[:M]
    return (gen_feat, class_pred)


# speedup vs baseline: 44.9556x; 1.6692x over previous
import functools

import jax
import jax.numpy as jnp
from jax import lax
from jax.experimental import pallas as pl
from jax.experimental.pallas import tpu as pltpu
from jax.experimental.pallas import tpu_sc as plsc

N = 10000
E = 320000
D = 128
T = 1000
NUM_PRED = 5
HID = 64
M = N + T * NUM_PRED
MPAD = 15232
NW = 32
EPB = 128
NB_E = E // EPB
TPAD = 5120
NBNEW = TPAD // EPB
RPT = MPAD // 16

C0_BPT = 102
C1_BASE = 1632

EV_BASE = 78
EV_SLAB = 79


def _core_split(s):
    lt2 = s < 2
    start = jnp.where(lt2, C1_BASE + s * 56, C1_BASE + 112 + (s - 2) * 54)
    off = jnp.where(lt2, 0, 2)
    npairs = jnp.where(lt2, 28, 27)
    return start - off, off, npairs


def _even_split(wid):
    start = wid * EV_BASE + jnp.minimum(wid, 4)
    cnt = EV_BASE + jnp.where(wid < 4, 1, 0)
    start_copy = jnp.minimum(start, NB_E - EV_SLAB)
    return start_copy, start - start_copy, cnt


@functools.cache
def _make_sc_count():
    mesh = plsc.VectorSubcoreMesh(core_axis_name="c", subcore_axis_name="s")
    return functools.partial(
        pl.kernel,
        mesh=mesh,
        compiler_params=pltpu.CompilerParams(
            needs_layout_passes=False, use_tc_tiling_on_sc=False),
        out_type=jax.ShapeDtypeStruct((NW, MPAD), jnp.float32),
        scratch_types=[
            pltpu.VMEM((EV_SLAB, 2, EPB), jnp.int32),
            pltpu.VMEM((MPAD,), jnp.float32),
        ],
    )(_sc_count_body)


def _sc_count_body(e3_hbm, out_hbm, slab, acc_v):
    c = lax.axis_index("c")
    s = lax.axis_index("s")
    wid = s * 2 + c
    start_copy, off, cnt = _even_split(wid)
    pltpu.sync_copy(e3_hbm.at[pl.ds(start_copy, EV_SLAB)], slab)

    def zero(i, carry):
        acc_v[pl.ds(i * 16, 16)] = jnp.zeros((16,), jnp.float32)
        return carry

    lax.fori_loop(0, MPAD // 16, zero, 0)
    ones = jnp.full((16,), 1.0, jnp.float32)

    def body(i, carry):
        j = off + i
        for k in range(EPB // 16):
            plsc.addupdate_scatter(acc_v, [slab[j, 1, pl.ds(k * 16, 16)]], ones)
        return carry

    lax.fori_loop(0, cnt, body, 0)
    pltpu.sync_copy(acc_v, out_hbm.at[wid])


@functools.cache
def _make_sc_scalar():
    mesh = plsc.VectorSubcoreMesh(core_axis_name="c", subcore_axis_name="s")
    return functools.partial(
        pl.kernel,
        mesh=mesh,
        compiler_params=pltpu.CompilerParams(
            needs_layout_passes=False, use_tc_tiling_on_sc=False),
        out_type=jax.ShapeDtypeStruct((NW, MPAD), jnp.float32),
        scratch_types=[
            pltpu.VMEM((EV_SLAB, 2, EPB), jnp.int32),
            pltpu.VMEM((NBNEW, EPB), jnp.int32),
            pltpu.VMEM((MPAD,), jnp.float32),
            pltpu.VMEM((MPAD,), jnp.float32),
        ],
    )(_sc_scalar_body)


def _sc_scalar_body(g2_hbm, e3_hbm, t3_hbm, out_hbm, slab, t3_v, g2_v, acc_v):
    c = lax.axis_index("c")
    s = lax.axis_index("s")
    wid = s * 2 + c
    start_copy, off, cnt = _even_split(wid)
    pltpu.sync_copy(e3_hbm.at[pl.ds(start_copy, EV_SLAB)], slab)
    pltpu.sync_copy(t3_hbm, t3_v)
    pltpu.sync_copy(g2_hbm, g2_v)

    def zero(i, carry):
        acc_v[pl.ds(i * 16, 16)] = jnp.zeros((16,), jnp.float32)
        return carry

    lax.fori_loop(0, MPAD // 16, zero, 0)

    def body(i, carry):
        j = off + i
        for k in range(EPB // 16):
            sl = pl.ds(k * 16, 16)
            vals = plsc.load_gather(g2_v, [slab[j, 0, sl]])
            plsc.addupdate_scatter(acc_v, [slab[j, 1, sl]], vals)
        return carry

    lax.fori_loop(0, cnt, body, 0)

    for extra in range(2):
        nb = wid + extra * NW

        @pl.when(nb < NBNEW)
        def _newblk():
            for k in range(EPB // 16):
                sl = pl.ds(k * 16, 16)
                vals = plsc.load_gather(g2_v, [t3_v[nb, sl]])
                acc_v[pl.ds(N + nb * EPB + k * 16, 16)] = vals

    pltpu.sync_copy(acc_v, out_hbm.at[wid])


@functools.cache
def _make_sc_rows():
    mesh = plsc.VectorSubcoreMesh(core_axis_name="c", subcore_axis_name="s")
    return functools.partial(
        pl.kernel,
        mesh=mesh,
        compiler_params=pltpu.CompilerParams(
            needs_layout_passes=False, use_tc_tiling_on_sc=False),
        out_type=jax.ShapeDtypeStruct((2, MPAD, HID), jnp.float32),
        scratch_types=[
            pltpu.VMEM((C0_BPT, 2, EPB), jnp.int32),
            pltpu.VMEM((56, 2, EPB), jnp.int32),
            pltpu.VMEM((NBNEW, EPB), jnp.int32),
            pltpu.VMEM((EPB, HID), jnp.float32),
            pltpu.VMEM((EPB, HID), jnp.float32),
            pltpu.SemaphoreType.DMA,
            pltpu.SemaphoreType.DMA,
            pltpu.VMEM_SHARED((MPAD, HID), jnp.float32),
        ],
    )(_sc_rows_body)


def _sc_rows_body(g_hbm, e3_hbm, t3_hbm, out_hbm,
                  slab0, slab1, t3_v, rows0, rows1, sem0, sem1, acc_sh):
    c = lax.axis_index("c")
    s = lax.axis_index("s")

    @pl.when(c == 0)
    def _stage0():
        pltpu.sync_copy(e3_hbm.at[pl.ds(s * C0_BPT, C0_BPT)], slab0)
        pltpu.sync_copy(t3_hbm, t3_v)

    start1, off1, npairs1 = _core_split(s)

    @pl.when(c == 1)
    def _stage1():
        pltpu.sync_copy(e3_hbm.at[pl.ds(start1, 56)], slab1)

    def zrow(r, carry):
        for k in range(HID // 16):
            rows0[r, pl.ds(k * 16, 16)] = jnp.zeros((16,), jnp.float32)
        return carry

    lax.fori_loop(0, EPB, zrow, 0)
    base = s * RPT
    nfull = RPT // EPB
    for k in range(nfull):
        pltpu.sync_copy(rows0, acc_sh.at[pl.ds(base + k * EPB, EPB)])
    rem = RPT - nfull * EPB
    if rem:
        pltpu.sync_copy(rows0.at[pl.ds(0, rem)],
                        acc_sh.at[pl.ds(base + nfull * EPB, rem)])
    plsc.subcore_barrier()

    @pl.when(c == 0)
    def _new_edges():
        for extra in range(3):
            nb = extra * 16 + s

            @pl.when(nb < NBNEW)
            def _newblk():
                pltpu.async_copy(g_hbm.at[t3_v.at[nb]], rows0, sem0).wait()
                pltpu.sync_copy(rows0, acc_sh.at[pl.ds(N + nb * EPB, EPB)])

    bufs = (rows0, rows1)
    sems = (sem0, sem1)

    def run_pairs(slab, off, npairs):
        pltpu.async_copy(g_hbm.at[slab.at[off, 0]], rows0, sem0)
        pltpu.async_copy(g_hbm.at[slab.at[off + 1, 0]], rows1, sem1)

        def body(i, carry):
            for b in range(2):
                j = off + 2 * i + b
                buf, sem = bufs[b], sems[b]
                pltpu.make_async_copy(g_hbm.at[slab.at[j, 0]], buf, sem).wait()
                pltpu.sync_copy(buf, acc_sh.at[slab.at[j, 1]], add=True)

                @pl.when(2 * i + b + 2 < 2 * npairs)
                def _prefetch():
                    pltpu.async_copy(g_hbm.at[slab.at[j + 2, 0]], buf, sem)

            return carry

        lax.fori_loop(0, npairs, body, 0)

    @pl.when(c == 0)
    def _main0():
        run_pairs(slab0, 0, C0_BPT // 2)

    @pl.when(c == 1)
    def _main1():
        run_pairs(slab1, off1, npairs1)

    plsc.subcore_barrier()
    pltpu.sync_copy(acc_sh.at[pl.ds(base, RPT)], out_hbm.at[c, pl.ds(base, RPT)])


def _mlp_body(feat_ref, w1_ref, b1_ref, w2_ref, b2_ref, wf_ref, bf_ref,
              wc1_ref, gen_ref, geng_ref):
    h = jnp.dot(feat_ref[...], w1_ref[...], preferred_element_type=jnp.float32)
    h = jnp.maximum(h + b1_ref[...], 0.0)
    h = jnp.dot(h, w2_ref[...], preferred_element_type=jnp.float32)
    h = jnp.maximum(h + b2_ref[...], 0.0)
    gen = jnp.tanh(jnp.dot(h, wf_ref[...], preferred_element_type=jnp.float32)
                   + bf_ref[...])
    gen_ref[...] = gen
    wc1 = wc1_ref[...]
    parts = [
        jnp.dot(gen[:, p * D:(p + 1) * D], wc1, preferred_element_type=jnp.float32)
        for p in range(NUM_PRED)
    ]
    geng_ref[...] = jnp.concatenate(parts, axis=1)


def _deg_body(parts_ref, out_ref):
    deg = jnp.sum(parts_ref[...], axis=0, keepdims=True) + 1.0
    col = lax.broadcasted_iota(jnp.int32, (1, MPAD), 1)
    deg = jnp.where(col < N, deg, 2.0)
    out_ref[...] = lax.rsqrt(deg)


def _prep_body(x_ref, wc1_ref, gen5_ref, dinv_ref, g_ref):
    hx = jnp.dot(x_ref[...], wc1_ref[...], preferred_element_type=jnp.float32)
    g_ref[pl.ds(0, N), :] = hx * dinv_ref[pl.ds(0, N), :]
    g_ref[pl.ds(N, M - N), :] = gen5_ref[...] * dinv_ref[pl.ds(N, M - N), :]
    g_ref[pl.ds(M, MPAD - M), :] = jnp.zeros((MPAD - M, HID), jnp.float32)


def _mid_body(acc_ref, g_ref, dinv_ref, bc1_ref, wc2_ref, g2_ref):
    t = acc_ref[0] + acc_ref[1] + g_ref[...]
    out1 = jnp.maximum(t * dinv_ref[...] + bc1_ref[...], 0.0)
    s = jnp.sum(out1 * wc2_ref[...], axis=1, keepdims=True)
    g2_ref[...] = s * dinv_ref[...]


def _final_body(parts_ref, g2_ref, dinv_ref, bc2_ref, out_ref):
    acc2 = jnp.sum(parts_ref[...], axis=0, keepdims=True)
    out2 = dinv_ref[...] * (acc2 + g2_ref[...]) + bc2_ref[...]
    out_ref[...] = 1.0 / (1.0 + jnp.exp(-out2))


def kernel(feat, x, edge_index, tails, W1, b1, W2, b2, Wf, bf, Wc1, bc1, Wc2, bc2):
    e3 = edge_index.reshape(2, NB_E, EPB).transpose(1, 0, 2)
    tails_rep = jnp.broadcast_to(tails[:, None], (T, NUM_PRED)).reshape(-1)
    t3 = jnp.concatenate(
        [tails_rep, jnp.zeros((TPAD - T * NUM_PRED,), jnp.int32)]
    ).reshape(NBNEW, EPB)

    parts = _make_sc_count()(e3)

    dinv_row = pl.pallas_call(
        _deg_body,
        out_shape=jax.ShapeDtypeStruct((1, MPAD), jnp.float32),
    )(parts)
    dinv_col = dinv_row.reshape(MPAD, 1)

    gen_feat, geng = pl.pallas_call(
        _mlp_body,
        out_shape=(
            jax.ShapeDtypeStruct((T, NUM_PRED * D), jnp.float32),
            jax.ShapeDtypeStruct((T, NUM_PRED * HID), jnp.float32),
        ),
    )(feat, W1, b1.reshape(1, -1), W2, b2.reshape(1, -1), Wf, bf.reshape(1, -1),
      Wc1)

    gen5 = geng.reshape(T * NUM_PRED, HID)

    g = pl.pallas_call(
        _prep_body,
        out_shape=jax.ShapeDtypeStruct((MPAD, HID), jnp.float32),
    )(x, Wc1, gen5, dinv_col)

    acc = _make_sc_rows()(g, e3, t3)

    g2_col = pl.pallas_call(
        _mid_body,
        out_shape=jax.ShapeDtypeStruct((MPAD, 1), jnp.float32),
    )(acc, g, dinv_col, bc1.reshape(1, -1), Wc2.reshape(1, -1))

    parts2 = _make_sc_scalar()(g2_col.reshape(MPAD), e3, t3)

    pred_row = pl.pallas_call(
        _final_body,
        out_shape=jax.ShapeDtypeStruct((1, MPAD), jnp.float32),
    )(parts2, g2_col.reshape(1, MPAD), dinv_row, bc2.reshape(1, 1))

    class_pred = pred_row.reshape(MPAD, 1)[:M]
    return (gen_feat, class_pred)
